# Initial kernel scaffold; baseline (speedup 1.0000x reference)
import jax
import jax.numpy as jnp
from jax.experimental import pallas as pl


def kernel(U, V, edge, negative_edges):
    raise NotImplementedError("write your pallas kernel here")


# speedup vs baseline: 1.9465x; 1.9465x over previous
import functools

import jax
import jax.numpy as jnp
from jax import lax
from jax.experimental import pallas as pl
from jax.experimental.pallas import tpu as pltpu
from jax.experimental.pallas import tpu_sc as plsc

_D = 128
_BATCH = 4096
_NNEG = 20
_EPS2 = 1e-16

_NW = 32
_PAIRS = _BATCH * (1 + _NNEG)
_PER_W = _PAIRS // _NW
_CHUNK = 128
_NCHUNK = _PER_W // _CHUNK
_L = 16


def _rsqrt(x):
    i = lax.bitcast_convert_type(x, jnp.int32)
    y = lax.bitcast_convert_type(jnp.int32(0x5F3759DF) - (i >> 1), jnp.float32)
    xh = 0.5 * x
    for _ in range(4):
        y = y * (1.5 - xh * y * y)
    return y


def _sc_body(uidx, vidx, u_tab, v_tab, out,
             idxu_v, idxv_v, ru_v, rv_v, suv_v, suu_v, svv_v, stage_v,
             semu, semv):
    wid = lax.axis_index("s") * 2 + lax.axis_index("c")
    pltpu.sync_copy(uidx.at[wid], idxu_v)
    pltpu.sync_copy(vidx.at[wid], idxv_v)
    lane = lax.iota(jnp.int32, _L)
    m_last = lane == (_L - 1)

    def chunk_body(c, acc):
        cu = pltpu.async_copy(u_tab.at[idxu_v.at[c]], ru_v, semu)
        cv = pltpu.async_copy(v_tab.at[idxv_v.at[c]], rv_v, semv)
        cu.wait()
        cv.wait()

        def pair_body(p, carry):
            uv = jnp.zeros((_L,), jnp.float32)
            uu = jnp.zeros((_L,), jnp.float32)
            vv = jnp.zeros((_L,), jnp.float32)
            for d in range(_D // _L):
                u = ru_v[p, pl.ds(d * _L, _L)]
                v = rv_v[p, pl.ds(d * _L, _L)]
                uv += u * v
                uu += u * u
                vv += v * v
            plsc.store_compressed(suv_v.at[pl.ds(p, _L)], plsc.cumsum(uv), mask=m_last)
            plsc.store_compressed(suu_v.at[pl.ds(p, _L)], plsc.cumsum(uu), mask=m_last)
            plsc.store_compressed(svv_v.at[pl.ds(p, _L)], plsc.cumsum(vv), mask=m_last)
            return carry

        lax.fori_loop(0, _CHUNK, pair_body, 0, unroll=2)

        base0 = wid * _PER_W + c * _CHUNK
        for g in range(_CHUNK // _L):
            suv = suv_v[pl.ds(g * _L, _L)]
            suu = suu_v[pl.ds(g * _L, _L)]
            svv = svv_v[pl.ds(g * _L, _L)]
            cos = suv * _rsqrt(jnp.maximum(suu, _EPS2)) * _rsqrt(jnp.maximum(svv, _EPS2))
            gid = base0 + g * _L + lane
            term = jnp.where(gid < _BATCH, 1.0 - cos,
                             jnp.maximum(cos - 1.0, 0.0))
            acc = acc + term
        return acc

    acc = lax.fori_loop(0, _NCHUNK, chunk_body, jnp.zeros((_L,), jnp.float32))
    stage_v[...] = acc
    pltpu.sync_copy(stage_v, out.at[wid])


_sc_call = functools.partial(
    pl.kernel,
    out_type=jax.ShapeDtypeStruct((_NW, _L), jnp.float32),
    mesh=plsc.VectorSubcoreMesh(core_axis_name="c", subcore_axis_name="s"),
    compiler_params=pltpu.CompilerParams(needs_layout_passes=False),
    scratch_types=[
        pltpu.VMEM((_NCHUNK, _CHUNK), jnp.int32),
        pltpu.VMEM((_NCHUNK, _CHUNK), jnp.int32),
        pltpu.VMEM((_CHUNK, _D), jnp.float32),
        pltpu.VMEM((_CHUNK, _D), jnp.float32),
        pltpu.VMEM((_CHUNK + _L,), jnp.float32),
        pltpu.VMEM((_CHUNK + _L,), jnp.float32),
        pltpu.VMEM((_CHUNK + _L,), jnp.float32),
        pltpu.VMEM((_L,), jnp.float32),
        pltpu.SemaphoreType.DMA,
        pltpu.SemaphoreType.DMA,
    ],
)(_sc_body)


def kernel(U, V, edge, negative_edges):
    edge32 = edge.astype(jnp.int32)
    neg32 = negative_edges.astype(jnp.int32)
    uidx = jnp.concatenate([edge32[:, 0], neg32[..., 0].reshape(-1)])
    vidx = jnp.concatenate([edge32[:, 1], neg32[..., 1].reshape(-1)])
    partials = _sc_call(uidx.reshape(_NW, _NCHUNK, _CHUNK),
                        vidx.reshape(_NW, _NCHUNK, _CHUNK), U, V)
    return -jnp.sum(partials)
